# baseline (speedup 1.0000x reference)
import jax
import jax.numpy as jnp
from jax.experimental import pallas as pl
from jax.experimental.pallas import tpu as pltpu

N = 4096
IN_FEAT = 64
OUT_FEAT = 64
BM = 256
T = N // BM
D = 12


def _copy(adj_hbm, bufs, sems, t):
    return pltpu.make_async_copy(
        adj_hbm.at[pl.ds(t * BM, BM), :], bufs.at[t % D], sems.at[t % D]
    )


def _kern(feat_ref, w_ref, adj_hbm, out_ref, x_ref, bufs, sems):
    for t in range(D):
        _copy(adj_hbm, bufs, sems, t).start()

    x_ref[...] = jnp.dot(
        feat_ref[...], w_ref[...], preferred_element_type=jnp.float32
    )
    x = x_ref[...]

    for t in range(T):
        _copy(adj_hbm, bufs, sems, t).wait()
        out_ref[pl.ds(t * BM, BM), :] = jnp.dot(
            bufs[t % D], x, preferred_element_type=jnp.float32
        )
        if t + D < T:
            _copy(adj_hbm, bufs, sems, t + D).start()


@jax.jit
def kernel(feat, adj, weight2):
    return pl.pallas_call(
        _kern,
        in_specs=[
            pl.BlockSpec(memory_space=pltpu.VMEM),
            pl.BlockSpec(memory_space=pltpu.VMEM),
            pl.BlockSpec(memory_space=pltpu.HBM),
        ],
        out_specs=pl.BlockSpec(memory_space=pltpu.VMEM),
        out_shape=jax.ShapeDtypeStruct((N, OUT_FEAT), jnp.float32),
        scratch_shapes=[
            pltpu.VMEM((N, OUT_FEAT), jnp.float32),
            pltpu.VMEM((D, BM, N), jnp.float32),
            pltpu.SemaphoreType.DMA((D,)),
        ],
        compiler_params=pltpu.CompilerParams(
            vmem_limit_bytes=63 * 1024 * 1024,
        ),
    )(feat, weight2, adj)
